# baseline (speedup 1.0000x reference)
import jax
import jax.numpy as jnp
from jax.experimental import pallas as pl

N = 10000; E = 320000; B = 64; FEAT = 128; SUB = 256; MOL = 512; NF = 100000
NH1 = 5; H1 = 128; NH2 = 1; O2 = 256; SLOPE = 0.1


def _gat_layer(h, src, dst, W, al, ar, b, R, nhead, dout, act):
    n = h.shape[0]
    feat = (h @ W).reshape(n, nhead, dout)
    el = (feat * al[None]).sum(-1)
    er = (feat * ar[None]).sum(-1)
    e = jax.nn.leaky_relu(el[src] + er[dst], SLOPE)
    m = jax.ops.segment_max(e, dst, num_segments=n)
    m = jnp.where(jnp.isfinite(m), m, 0.0)
    ex = jnp.exp(e - m[dst])
    s = jax.ops.segment_sum(ex, dst, num_segments=n)
    alpha = ex / (s[dst] + 1e-9)
    out = jax.ops.segment_sum(feat[src] * alpha[..., None], dst, num_segments=n)
    out = out.reshape(n, nhead * dout) + b + h @ R
    if act:
        out = jax.nn.relu(out)
    return out


def _gin_layer(h, src, dst, w1, b1, w2, b2):
    agg = jax.ops.segment_sum(h[src], dst, num_segments=h.shape[0])
    z = h + agg
    z = jax.nn.relu(z @ w1 + b1) @ w2 + b2
    return z + h


def _head_body(y_ref, p1_ref, p2_ref, p3_ref, o_ref):
    z = jnp.maximum(y_ref[...] @ p1_ref[...], 0.0)
    z = jnp.maximum(z @ p2_ref[...], 0.0)
    o_ref[...] = z @ p3_ref[...]


def kernel(node_feature, h_MolCLR, maccs, morgan, params, edge_index, node_subgraph, graph_ids):
    src, dst = edge_index[0], edge_index[1]
    p = params
    x_sub = p['embed'][node_subgraph]
    x_sub = _gin_layer(x_sub, src, dst, p['gin_w1_0'], p['gin_b1_0'], p['gin_w2_0'], p['gin_b2_0'])
    x_sub = _gin_layer(x_sub, src, dst, p['gin_w1_1'], p['gin_b1_1'], p['gin_w2_1'], p['gin_b2_1'])
    x_sub_pool = jax.ops.segment_sum(x_sub, graph_ids, num_segments=B)
    x_feat = _gat_layer(node_feature, src, dst, p['gat_W_0'], p['gat_al_0'], p['gat_ar_0'], p['gat_b_0'], p['gat_R_0'], NH1, H1, True)
    x_feat = _gat_layer(x_feat, src, dst, p['gat_W_1'], p['gat_al_1'], p['gat_ar_1'], p['gat_b_1'], p['gat_R_1'], NH2, O2, False)
    cnt = jax.ops.segment_sum(jnp.ones((x_feat.shape[0],), jnp.float32), graph_ids, num_segments=B)
    x_feat_pool = jax.ops.segment_sum(x_feat, graph_ids, num_segments=B) / jnp.maximum(cnt, 1.0)[:, None]
    hM = h_MolCLR @ p['adM_w'] + p['adM_b']
    fp = jnp.concatenate([maccs, morgan], axis=1).astype(jnp.float32) @ p['adF_w'] + p['adF_b']
    y = jnp.concatenate([x_sub_pool, x_feat_pool], axis=1) + hM + fp
    out = pl.pallas_call(
        _head_body,
        out_shape=jax.ShapeDtypeStruct((B, 11), jnp.float32),
    )(y, p['p1'], p['p2'], p['p3'])
    return out


# speedup vs baseline: 1.0257x; 1.0257x over previous
import functools

import jax
import jax.numpy as jnp
from jax import lax
from jax.experimental import pallas as pl
from jax.experimental.pallas import tpu as pltpu
from jax.experimental.pallas import tpu_sc as plsc

N = 10000; E = 320000; B = 64; FEAT = 128; SUB = 256; MOL = 512; NF = 100000
NH1 = 5; H1 = 128; NH2 = 1; O2 = 256; SLOPE = 0.1

NP = 10240
EK = 80
_MESH = plsc.VectorSubcoreMesh(core_axis_name="c", subcore_axis_name="s")


def _seg_sum_128(table, src, dst, col0, width):
    e_total = src.shape[0]
    per_w = e_total // 32
    nblk = per_w // EK
    rows_per_sub = NP // 16

    @functools.partial(
        pl.kernel,
        mesh=_MESH,
        out_type=jax.ShapeDtypeStruct((2, NP, width), jnp.float32),
        scratch_types=[
            pltpu.VMEM((EK,), jnp.int32),
            pltpu.VMEM((EK,), jnp.int32),
            pltpu.VMEM((EK, width), jnp.float32),
            pltpu.VMEM_SHARED((NP, width), jnp.float32),
            pltpu.SemaphoreType.DMA,
        ],
    )
    def k(tab, src_h, dst_h, zeros_h, out, sidx, didx, rows, acc, sem):
        c = lax.axis_index("c")
        s = lax.axis_index("s")
        wid = s * 2 + c
        zbase = pl.multiple_of(s * rows_per_sub, 8)
        pltpu.sync_copy(zeros_h.at[pl.ds(zbase, rows_per_sub)],
                        acc.at[pl.ds(zbase, rows_per_sub)])
        plsc.subcore_barrier()

        def body(j, carry):
            base = pl.multiple_of(wid * per_w + j * EK, 8)
            pltpu.sync_copy(src_h.at[pl.ds(base, EK)], sidx)
            pltpu.sync_copy(dst_h.at[pl.ds(base, EK)], didx)
            pltpu.async_copy(tab.at[sidx, pl.ds(col0, width)], rows, sem).wait()
            pltpu.sync_copy(rows, acc.at[didx], add=True)
            return carry

        lax.fori_loop(0, nblk, body, 0)
        plsc.subcore_barrier()
        pltpu.sync_copy(acc.at[pl.ds(zbase, rows_per_sub)],
                        out.at[c, pl.ds(zbase, rows_per_sub)])

    zeros = jnp.zeros((NP, width), jnp.float32)
    return k(table, src, dst, zeros)


def _segment_sum_sc(h, src, dst):
    pl_ = _seg_sum_128(h, src, dst, 0, 128)
    pr_ = _seg_sum_128(h, src, dst, 128, 128)
    left = (pl_[0] + pl_[1])[:N]
    right = (pr_[0] + pr_[1])[:N]
    return jnp.concatenate([left, right], axis=1)


def _gat_layer(h, src, dst, W, al, ar, b, R, nhead, dout, act):
    n = h.shape[0]
    feat = (h @ W).reshape(n, nhead, dout)
    el = (feat * al[None]).sum(-1)
    er = (feat * ar[None]).sum(-1)
    e = jax.nn.leaky_relu(el[src] + er[dst], SLOPE)
    m = jax.ops.segment_max(e, dst, num_segments=n)
    m = jnp.where(jnp.isfinite(m), m, 0.0)
    ex = jnp.exp(e - m[dst])
    s = jax.ops.segment_sum(ex, dst, num_segments=n)
    alpha = ex / (s[dst] + 1e-9)
    out = jax.ops.segment_sum(feat[src] * alpha[..., None], dst, num_segments=n)
    out = out.reshape(n, nhead * dout) + b + h @ R
    if act:
        out = jax.nn.relu(out)
    return out


def _gin_layer(h, src, dst, w1, b1, w2, b2):
    agg = _segment_sum_sc(h, src, dst)
    z = h + agg
    z = jax.nn.relu(z @ w1 + b1) @ w2 + b2
    return z + h


def _head_body(y_ref, p1_ref, p2_ref, p3_ref, o_ref):
    z = jnp.maximum(y_ref[...] @ p1_ref[...], 0.0)
    z = jnp.maximum(z @ p2_ref[...], 0.0)
    o_ref[...] = z @ p3_ref[...]


def kernel(node_feature, h_MolCLR, maccs, morgan, params, edge_index, node_subgraph, graph_ids):
    src, dst = edge_index[0], edge_index[1]
    p = params
    x_sub = p['embed'][node_subgraph]
    x_sub = _gin_layer(x_sub, src, dst, p['gin_w1_0'], p['gin_b1_0'], p['gin_w2_0'], p['gin_b2_0'])
    x_sub = _gin_layer(x_sub, src, dst, p['gin_w1_1'], p['gin_b1_1'], p['gin_w2_1'], p['gin_b2_1'])
    x_sub_pool = jax.ops.segment_sum(x_sub, graph_ids, num_segments=B)
    x_feat = _gat_layer(node_feature, src, dst, p['gat_W_0'], p['gat_al_0'], p['gat_ar_0'], p['gat_b_0'], p['gat_R_0'], NH1, H1, True)
    x_feat = _gat_layer(x_feat, src, dst, p['gat_W_1'], p['gat_al_1'], p['gat_ar_1'], p['gat_b_1'], p['gat_R_1'], NH2, O2, False)
    cnt = jax.ops.segment_sum(jnp.ones((x_feat.shape[0],), jnp.float32), graph_ids, num_segments=B)
    x_feat_pool = jax.ops.segment_sum(x_feat, graph_ids, num_segments=B) / jnp.maximum(cnt, 1.0)[:, None]
    hM = h_MolCLR @ p['adM_w'] + p['adM_b']
    fp = jnp.concatenate([maccs, morgan], axis=1).astype(jnp.float32) @ p['adF_w'] + p['adF_b']
    y = jnp.concatenate([x_sub_pool, x_feat_pool], axis=1) + hM + fp
    out = pl.pallas_call(
        _head_body,
        out_shape=jax.ShapeDtypeStruct((B, 11), jnp.float32),
    )(y, p['p1'], p['p2'], p['p3'])
    return out


# speedup vs baseline: 9.1292x; 8.9007x over previous
import functools

import jax
import jax.numpy as jnp
from jax import lax
from jax.experimental import pallas as pl
from jax.experimental.pallas import tpu as pltpu
from jax.experimental.pallas import tpu_sc as plsc

N = 10000; E = 320000; B = 64; FEAT = 128; SUB = 256; MOL = 512; NF = 100000
NH1 = 5; H1 = 128; NH2 = 1; O2 = 256; SLOPE = 0.1

NP = 10240
EK = 80
_MESH = plsc.VectorSubcoreMesh(core_axis_name="c", subcore_axis_name="s")


def _seg_sum_128(table, src, dst, col0, width):
    e_total = src.shape[0]
    per_w = e_total // 32
    nblk = per_w // EK
    rows_per_sub = NP // 16

    @functools.partial(
        pl.kernel,
        mesh=_MESH,
        out_type=jax.ShapeDtypeStruct((2, NP, width), jnp.float32),
        scratch_types=[
            pltpu.VMEM((EK,), jnp.int32),
            pltpu.VMEM((EK,), jnp.int32),
            pltpu.VMEM((EK, width), jnp.float32),
            pltpu.VMEM_SHARED((NP, width), jnp.float32),
            pltpu.SemaphoreType.DMA,
        ],
    )
    def k(tab, src_h, dst_h, zeros_h, out, sidx, didx, rows, acc, sem):
        c = lax.axis_index("c")
        s = lax.axis_index("s")
        wid = s * 2 + c
        zbase = pl.multiple_of(s * rows_per_sub, 8)
        pltpu.sync_copy(zeros_h.at[pl.ds(zbase, rows_per_sub)],
                        acc.at[pl.ds(zbase, rows_per_sub)])
        plsc.subcore_barrier()

        def body(j, carry):
            base = pl.multiple_of(wid * per_w + j * EK, 8)
            pltpu.sync_copy(src_h.at[pl.ds(base, EK)], sidx)
            pltpu.sync_copy(dst_h.at[pl.ds(base, EK)], didx)
            pltpu.async_copy(tab.at[sidx, pl.ds(col0, width)], rows, sem).wait()
            pltpu.sync_copy(rows, acc.at[didx], add=True)
            return carry

        lax.fori_loop(0, nblk, body, 0)
        plsc.subcore_barrier()
        pltpu.sync_copy(acc.at[pl.ds(zbase, rows_per_sub)],
                        out.at[c, pl.ds(zbase, rows_per_sub)])

    zeros = jnp.zeros((NP, width), jnp.float32)
    return k(table, src, dst, zeros)


def _embed_gather(embed, nsub_pad):
    per_w = NP // 32
    nblk = per_w // EK

    @functools.partial(
        pl.kernel,
        mesh=_MESH,
        out_type=jax.ShapeDtypeStruct((NP, SUB), jnp.float32),
        scratch_types=[
            pltpu.VMEM((EK,), jnp.int32),
            pltpu.VMEM((EK, SUB), jnp.float32),
            pltpu.SemaphoreType.DMA,
        ],
    )
    def k(emb, nsub, out, idx, buf, sem):
        c = lax.axis_index("c")
        s = lax.axis_index("s")
        wid = s * 2 + c

        def body(j, carry):
            base = pl.multiple_of(wid * per_w + j * EK, 8)
            pltpu.sync_copy(nsub.at[pl.ds(base, EK)], idx)
            pltpu.async_copy(emb.at[idx], buf, sem).wait()
            pltpu.sync_copy(buf, out.at[pl.ds(base, EK)])
            return carry

        lax.fori_loop(0, nblk, body, 0)

    return k(embed, nsub_pad)


def _gat_edge_pass1(el128, er128, src, dst):
    per_w = E // 32
    nblk = per_w // EK
    rows_per_sub = NP // 16

    @functools.partial(
        pl.kernel,
        mesh=_MESH,
        out_type=[jax.ShapeDtypeStruct((E, 16), jnp.float32),
                  jax.ShapeDtypeStruct((2, NP, 128), jnp.float32)],
        scratch_types=[
            pltpu.VMEM((EK,), jnp.int32),
            pltpu.VMEM((EK,), jnp.int32),
            pltpu.VMEM((EK, 128), jnp.float32),
            pltpu.VMEM((EK, 128), jnp.float32),
            pltpu.VMEM((EK, 16), jnp.float32),
            pltpu.VMEM((EK, 128), jnp.float32),
            pltpu.VMEM_SHARED((NP, 128), jnp.float32),
            pltpu.SemaphoreType.DMA,
            pltpu.SemaphoreType.DMA,
        ],
    )
    def k(el, er, src_h, dst_h, zeros_h, ex_out, s_out,
          sidx, didx, abuf, bbuf, exb16, exb128, sacc, sem, sem2):
        c = lax.axis_index("c")
        s = lax.axis_index("s")
        wid = s * 2 + c
        zbase = pl.multiple_of(s * rows_per_sub, 8)
        pltpu.sync_copy(zeros_h.at[pl.ds(zbase, rows_per_sub)],
                        sacc.at[pl.ds(zbase, rows_per_sub)])
        pltpu.sync_copy(zeros_h.at[pl.ds(0, EK)], exb128)
        plsc.subcore_barrier()

        def body(j, carry):
            base = pl.multiple_of(wid * per_w + j * EK, 8)
            pltpu.sync_copy(src_h.at[pl.ds(base, EK)], sidx)
            pltpu.sync_copy(dst_h.at[pl.ds(base, EK)], didx)
            cp_a = pltpu.async_copy(el.at[sidx], abuf, sem)
            cp_b = pltpu.async_copy(er.at[didx], bbuf, sem2)
            cp_a.wait()
            cp_b.wait()

            def erow(i, carry2):
                v = abuf[i, pl.ds(0, 16)] + bbuf[i, pl.ds(0, 16)]
                v = jnp.maximum(v, SLOPE * v)
                exv = jnp.exp(v)
                exb16[i, :] = exv
                exb128[i, pl.ds(0, 16)] = exv
                return carry2

            lax.fori_loop(0, EK, erow, 0)
            pltpu.sync_copy(exb16, ex_out.at[pl.ds(base, EK)])
            pltpu.sync_copy(exb128, sacc.at[didx], add=True)
            return carry

        lax.fori_loop(0, nblk, body, 0)
        plsc.subcore_barrier()
        pltpu.sync_copy(sacc.at[pl.ds(zbase, rows_per_sub)],
                        s_out.at[c, pl.ds(zbase, rows_per_sub)])

    zeros = jnp.zeros((NP, 128), jnp.float32)
    return k(el128, er128, src, dst, zeros)


def _gat_edge_pass2(feat, ex16, src, dst, col0, hcol):
    per_w = E // 32
    nblk = per_w // EK
    rows_per_sub = NP // 16
    width = 128

    @functools.partial(
        pl.kernel,
        mesh=_MESH,
        out_type=jax.ShapeDtypeStruct((2, NP, width), jnp.float32),
        scratch_types=[
            pltpu.VMEM((EK,), jnp.int32),
            pltpu.VMEM((EK,), jnp.int32),
            pltpu.VMEM((EK, width), jnp.float32),
            pltpu.VMEM((EK, 16), jnp.float32),
            pltpu.VMEM_SHARED((NP, width), jnp.float32),
            pltpu.SemaphoreType.DMA,
        ],
    )
    def k(feat_h, ex_h, src_h, dst_h, zeros_h, out,
          sidx, didx, rows, exb, acc, sem):
        c = lax.axis_index("c")
        s = lax.axis_index("s")
        wid = s * 2 + c
        zbase = pl.multiple_of(s * rows_per_sub, 8)
        pltpu.sync_copy(zeros_h.at[pl.ds(zbase, rows_per_sub)],
                        acc.at[pl.ds(zbase, rows_per_sub)])
        plsc.subcore_barrier()

        def body(j, carry):
            base = pl.multiple_of(wid * per_w + j * EK, 8)
            pltpu.sync_copy(src_h.at[pl.ds(base, EK)], sidx)
            pltpu.sync_copy(dst_h.at[pl.ds(base, EK)], didx)
            pltpu.sync_copy(ex_h.at[pl.ds(base, EK)], exb)
            pltpu.async_copy(feat_h.at[sidx, pl.ds(col0, width)], rows, sem).wait()

            def escale(i, carry2):
                exv = exb[i, :]
                sc = jnp.full((16,), exv[hcol], jnp.float32)
                for jj in range(width // 16):
                    rows[i, pl.ds(jj * 16, 16)] = rows[i, pl.ds(jj * 16, 16)] * sc
                return carry2

            lax.fori_loop(0, EK, escale, 0)
            pltpu.sync_copy(rows, acc.at[didx], add=True)
            return carry

        lax.fori_loop(0, nblk, body, 0)
        plsc.subcore_barrier()
        pltpu.sync_copy(acc.at[pl.ds(zbase, rows_per_sub)],
                        out.at[c, pl.ds(zbase, rows_per_sub)])

    zeros = jnp.zeros((NP, width), jnp.float32)
    return k(feat, ex16, src, dst, zeros)


def _pool(parts, gidp, onesb):
    per_w = NP // 32
    nblk = per_w // EK
    W = 128

    @functools.partial(
        pl.kernel,
        mesh=_MESH,
        out_type=[jax.ShapeDtypeStruct((2, 128, W), jnp.float32)] * 5,
        scratch_types=(
            [pltpu.VMEM((EK,), jnp.int32)]
            + [pltpu.VMEM((EK, W), jnp.float32)] * 5
            + [pltpu.VMEM_SHARED((128, W), jnp.float32)] * 5
            + [pltpu.SemaphoreType.DMA]
        ),
    )
    def k(a0, a1, a2, a3, gid, ones_h, zero_h,
          o0, o1, o2, o3, o4,
          gidx, b0, b1, b2, b3, bc, c0, c1, c2, c3, c4, sem):
        c = lax.axis_index("c")
        s = lax.axis_index("s")
        wid = s * 2 + c
        zb = pl.multiple_of(s * 8, 8)
        for acc in (c0, c1, c2, c3, c4):
            pltpu.sync_copy(zero_h.at[pl.ds(zb, 8)], acc.at[pl.ds(zb, 8)])
        pltpu.sync_copy(ones_h, bc)
        plsc.subcore_barrier()

        def body(j, carry):
            base = pl.multiple_of(wid * per_w + j * EK, 8)
            pltpu.sync_copy(gid.at[pl.ds(base, EK)], gidx)
            for src_h, buf, acc in ((a0, b0, c0), (a1, b1, c1),
                                    (a2, b2, c2), (a3, b3, c3)):
                pltpu.sync_copy(src_h.at[pl.ds(base, EK)], buf)
                pltpu.sync_copy(buf, acc.at[gidx], add=True)
            pltpu.sync_copy(bc, c4.at[gidx], add=True)
            return carry

        lax.fori_loop(0, nblk, body, 0)
        plsc.subcore_barrier()
        for acc, out in ((c0, o0), (c1, o1), (c2, o2), (c3, o3), (c4, o4)):
            pltpu.sync_copy(acc.at[pl.ds(zb, 8)], out.at[c, pl.ds(zb, 8)])

    zero = jnp.zeros((128, W), jnp.float32)
    return k(parts[0], parts[1], parts[2], parts[3], gidp, onesb, zero)


def _segment_sum_sc(h, src, dst):
    pl_ = _seg_sum_128(h, src, dst, 0, 128)
    pr_ = _seg_sum_128(h, src, dst, 128, 128)
    left = (pl_[0] + pl_[1])[:N]
    right = (pr_[0] + pr_[1])[:N]
    return jnp.concatenate([left, right], axis=1)


def _gin_layer(h, src, dst, w1, b1, w2, b2):
    agg = _segment_sum_sc(h, src, dst)
    z = h + agg
    z = jax.nn.relu(z @ w1 + b1) @ w2 + b2
    return z + h


def _gat_layer_sc(h, src, dst, W, al, ar, b, R, nhead, dout, act):
    n = h.shape[0]
    feat = h @ W
    feat3 = feat.reshape(n, nhead, dout)
    el = (feat3 * al[None]).sum(-1)
    er = (feat3 * ar[None]).sum(-1)
    el128 = jnp.pad(el, ((0, 0), (0, 128 - nhead)))
    er128 = jnp.pad(er, ((0, 0), (0, 128 - nhead)))
    ex16, s_part = _gat_edge_pass1(el128, er128, src, dst)
    s = (s_part[0] + s_part[1])[:N, :nhead]
    num = []
    for hd in range(nhead):
        for half in range(dout // 128):
            part = _gat_edge_pass2(feat, ex16, src, dst, hd * dout + half * 128, hd)
            num.append((part[0] + part[1])[:N])
    o = jnp.concatenate(num, axis=1).reshape(n, nhead, dout)
    o = o / (s[..., None] + 1e-9)
    out = o.reshape(n, nhead * dout) + b + h @ R
    if act:
        out = jax.nn.relu(out)
    return out


def _head_body(y_ref, p1_ref, p2_ref, p3_ref, o_ref):
    z = jnp.maximum(y_ref[...] @ p1_ref[...], 0.0)
    z = jnp.maximum(z @ p2_ref[...], 0.0)
    o_ref[...] = z @ p3_ref[...]


def kernel(node_feature, h_MolCLR, maccs, morgan, params, edge_index, node_subgraph, graph_ids):
    src, dst = edge_index[0], edge_index[1]
    p = params
    nsub_pad = jnp.pad(node_subgraph, (0, NP - N))
    x_sub = _embed_gather(p['embed'], nsub_pad)[:N]
    x_sub = _gin_layer(x_sub, src, dst, p['gin_w1_0'], p['gin_b1_0'], p['gin_w2_0'], p['gin_b2_0'])
    x_sub = _gin_layer(x_sub, src, dst, p['gin_w1_1'], p['gin_b1_1'], p['gin_w2_1'], p['gin_b2_1'])
    x_feat = _gat_layer_sc(node_feature, src, dst, p['gat_W_0'], p['gat_al_0'], p['gat_ar_0'], p['gat_b_0'], p['gat_R_0'], NH1, H1, True)
    x_feat = _gat_layer_sc(x_feat, src, dst, p['gat_W_1'], p['gat_al_1'], p['gat_ar_1'], p['gat_b_1'], p['gat_R_1'], NH2, O2, False)

    gidp = jnp.pad(graph_ids.astype(jnp.int32), (0, NP - N), constant_values=64)
    x2p = jnp.pad(x_sub, ((0, NP - N), (0, 0)))
    xf2p = jnp.pad(x_feat, ((0, NP - N), (0, 0)))
    onesb = jnp.ones((EK, 128), jnp.float32)
    parts = (x2p[:, :128], x2p[:, 128:], xf2p[:, :128], xf2p[:, 128:])
    o0, o1, o2, o3, o4 = _pool(parts, gidp, onesb)
    x_sub_pool = jnp.concatenate(
        [(o0[0] + o0[1])[:B], (o1[0] + o1[1])[:B]], axis=1)
    cnt = (o4[0] + o4[1])[:B, 0]
    x_feat_pool = jnp.concatenate(
        [(o2[0] + o2[1])[:B], (o3[0] + o3[1])[:B]], axis=1) / jnp.maximum(cnt, 1.0)[:, None]

    hM = h_MolCLR @ p['adM_w'] + p['adM_b']
    fp = jnp.concatenate([maccs, morgan], axis=1).astype(jnp.float32) @ p['adF_w'] + p['adF_b']
    y = jnp.concatenate([x_sub_pool, x_feat_pool], axis=1) + hM + fp
    out = pl.pallas_call(
        _head_body,
        out_shape=jax.ShapeDtypeStruct((B, 11), jnp.float32),
    )(y, p['p1'], p['p2'], p['p3'])
    return out


# speedup vs baseline: 9.2417x; 1.0123x over previous
import functools

import jax
import jax.numpy as jnp
from jax import lax
from jax.experimental import pallas as pl
from jax.experimental.pallas import tpu as pltpu
from jax.experimental.pallas import tpu_sc as plsc

N = 10000; E = 320000; B = 64; FEAT = 128; SUB = 256; MOL = 512; NF = 100000
NH1 = 5; H1 = 128; NH2 = 1; O2 = 256; SLOPE = 0.1

NP = 10240
EK = 80
_MESH = plsc.VectorSubcoreMesh(core_axis_name="c", subcore_axis_name="s")


def _seg_sum_128(table, src, dst, col0, width):
    e_total = src.shape[0]
    per_w = e_total // 32
    nblk = per_w // EK
    rows_per_sub = NP // 16

    @functools.partial(
        pl.kernel,
        mesh=_MESH,
        out_type=jax.ShapeDtypeStruct((2, NP, width), jnp.float32),
        scratch_types=[
            pltpu.VMEM((EK,), jnp.int32),
            pltpu.VMEM((EK,), jnp.int32),
            pltpu.VMEM((EK, width), jnp.float32),
            pltpu.VMEM_SHARED((NP, width), jnp.float32),
            pltpu.SemaphoreType.DMA,
        ],
    )
    def k(tab, src_h, dst_h, zeros_h, out, sidx, didx, rows, acc, sem):
        c = lax.axis_index("c")
        s = lax.axis_index("s")
        wid = s * 2 + c
        zbase = pl.multiple_of(s * rows_per_sub, 8)
        pltpu.sync_copy(zeros_h.at[pl.ds(zbase, rows_per_sub)],
                        acc.at[pl.ds(zbase, rows_per_sub)])
        plsc.subcore_barrier()

        def body(j, carry):
            base = pl.multiple_of(wid * per_w + j * EK, 8)
            pltpu.sync_copy(src_h.at[pl.ds(base, EK)], sidx)
            pltpu.sync_copy(dst_h.at[pl.ds(base, EK)], didx)
            pltpu.async_copy(tab.at[sidx, pl.ds(col0, width)], rows, sem).wait()
            pltpu.sync_copy(rows, acc.at[didx], add=True)
            return carry

        lax.fori_loop(0, nblk, body, 0)
        plsc.subcore_barrier()
        pltpu.sync_copy(acc.at[pl.ds(zbase, rows_per_sub)],
                        out.at[c, pl.ds(zbase, rows_per_sub)])

    zeros = jnp.zeros((NP, width), jnp.float32)
    return k(table, src, dst, zeros)


def _embed_gather(embed, nsub_pad):
    per_w = NP // 32
    nblk = per_w // EK

    @functools.partial(
        pl.kernel,
        mesh=_MESH,
        out_type=jax.ShapeDtypeStruct((NP, SUB), jnp.float32),
        scratch_types=[
            pltpu.VMEM((EK,), jnp.int32),
            pltpu.VMEM((EK, SUB), jnp.float32),
            pltpu.SemaphoreType.DMA,
        ],
    )
    def k(emb, nsub, out, idx, buf, sem):
        c = lax.axis_index("c")
        s = lax.axis_index("s")
        wid = s * 2 + c

        def body(j, carry):
            base = pl.multiple_of(wid * per_w + j * EK, 8)
            pltpu.sync_copy(nsub.at[pl.ds(base, EK)], idx)
            pltpu.async_copy(emb.at[idx], buf, sem).wait()
            pltpu.sync_copy(buf, out.at[pl.ds(base, EK)])
            return carry

        lax.fori_loop(0, nblk, body, 0)

    return k(embed, nsub_pad)


def _gat_edge_pass1(el128, er128, src, dst):
    per_w = E // 32
    nblk = per_w // EK
    rows_per_sub = NP // 16

    @functools.partial(
        pl.kernel,
        mesh=_MESH,
        out_type=[jax.ShapeDtypeStruct((E, 16), jnp.float32),
                  jax.ShapeDtypeStruct((2, NP, 128), jnp.float32)],
        scratch_types=[
            pltpu.VMEM((EK,), jnp.int32),
            pltpu.VMEM((EK,), jnp.int32),
            pltpu.VMEM((EK, 128), jnp.float32),
            pltpu.VMEM((EK, 128), jnp.float32),
            pltpu.VMEM((EK, 16), jnp.float32),
            pltpu.VMEM((EK, 128), jnp.float32),
            pltpu.VMEM_SHARED((NP, 128), jnp.float32),
            pltpu.SemaphoreType.DMA,
            pltpu.SemaphoreType.DMA,
        ],
    )
    def k(el, er, src_h, dst_h, zeros_h, ex_out, s_out,
          sidx, didx, abuf, bbuf, exb16, exb128, sacc, sem, sem2):
        c = lax.axis_index("c")
        s = lax.axis_index("s")
        wid = s * 2 + c
        zbase = pl.multiple_of(s * rows_per_sub, 8)
        pltpu.sync_copy(zeros_h.at[pl.ds(zbase, rows_per_sub)],
                        sacc.at[pl.ds(zbase, rows_per_sub)])
        pltpu.sync_copy(zeros_h.at[pl.ds(0, EK)], exb128)
        plsc.subcore_barrier()

        def body(j, carry):
            base = pl.multiple_of(wid * per_w + j * EK, 8)
            pltpu.sync_copy(src_h.at[pl.ds(base, EK)], sidx)
            pltpu.sync_copy(dst_h.at[pl.ds(base, EK)], didx)
            cp_a = pltpu.async_copy(el.at[sidx], abuf, sem)
            cp_b = pltpu.async_copy(er.at[didx], bbuf, sem2)
            cp_a.wait()
            cp_b.wait()

            def erow(i, carry2):
                v = abuf[i, pl.ds(0, 16)] + bbuf[i, pl.ds(0, 16)]
                v = jnp.maximum(v, SLOPE * v)
                exv = jnp.exp(v)
                exb16[i, :] = exv
                exb128[i, pl.ds(0, 16)] = exv
                return carry2

            lax.fori_loop(0, EK, erow, 0)
            pltpu.sync_copy(exb16, ex_out.at[pl.ds(base, EK)])
            pltpu.sync_copy(exb128, sacc.at[didx], add=True)
            return carry

        lax.fori_loop(0, nblk, body, 0)
        plsc.subcore_barrier()
        pltpu.sync_copy(sacc.at[pl.ds(zbase, rows_per_sub)],
                        s_out.at[c, pl.ds(zbase, rows_per_sub)])

    zeros = jnp.zeros((NP, 128), jnp.float32)
    return k(el128, er128, src, dst, zeros)


def _gat_edge_pass2(feat, ex16, src, dst, col0, hcol):
    per_w = E // 32
    nblk = per_w // EK
    rows_per_sub = NP // 16
    width = 128

    @functools.partial(
        pl.kernel,
        mesh=_MESH,
        out_type=jax.ShapeDtypeStruct((2, NP, width), jnp.float32),
        scratch_types=[
            pltpu.VMEM((EK,), jnp.int32),
            pltpu.VMEM((EK,), jnp.int32),
            pltpu.VMEM((EK, width), jnp.float32),
            pltpu.VMEM((EK, 16), jnp.float32),
            pltpu.VMEM_SHARED((NP, width), jnp.float32),
            pltpu.SemaphoreType.DMA,
        ],
    )
    def k(feat_h, ex_h, src_h, dst_h, zeros_h, out,
          sidx, didx, rows, exb, acc, sem):
        c = lax.axis_index("c")
        s = lax.axis_index("s")
        wid = s * 2 + c
        zbase = pl.multiple_of(s * rows_per_sub, 8)
        pltpu.sync_copy(zeros_h.at[pl.ds(zbase, rows_per_sub)],
                        acc.at[pl.ds(zbase, rows_per_sub)])
        plsc.subcore_barrier()

        def body(j, carry):
            base = pl.multiple_of(wid * per_w + j * EK, 8)
            pltpu.sync_copy(src_h.at[pl.ds(base, EK)], sidx)
            pltpu.sync_copy(dst_h.at[pl.ds(base, EK)], didx)
            pltpu.sync_copy(ex_h.at[pl.ds(base, EK)], exb)
            pltpu.async_copy(feat_h.at[sidx, pl.ds(col0, width)], rows, sem).wait()

            def escale(i, carry2):
                exv = exb[i, :]
                sc = jnp.full((16,), exv[hcol], jnp.float32)
                for jj in range(width // 16):
                    rows[i, pl.ds(jj * 16, 16)] = rows[i, pl.ds(jj * 16, 16)] * sc
                return carry2

            lax.fori_loop(0, EK, escale, 0)
            pltpu.sync_copy(rows, acc.at[didx], add=True)
            return carry

        lax.fori_loop(0, nblk, body, 0)
        plsc.subcore_barrier()
        pltpu.sync_copy(acc.at[pl.ds(zbase, rows_per_sub)],
                        out.at[c, pl.ds(zbase, rows_per_sub)])

    zeros = jnp.zeros((NP, width), jnp.float32)
    return k(feat, ex16, src, dst, zeros)


def _pool(parts, gidp, onesb):
    per_w = NP // 32
    nblk = per_w // EK
    W = 128

    @functools.partial(
        pl.kernel,
        mesh=_MESH,
        out_type=[jax.ShapeDtypeStruct((2, 128, W), jnp.float32)] * 5,
        scratch_types=(
            [pltpu.VMEM((EK,), jnp.int32)]
            + [pltpu.VMEM((EK, W), jnp.float32)] * 5
            + [pltpu.VMEM_SHARED((128, W), jnp.float32)] * 5
            + [pltpu.SemaphoreType.DMA]
        ),
    )
    def k(a0, a1, a2, a3, gid, ones_h, zero_h,
          o0, o1, o2, o3, o4,
          gidx, b0, b1, b2, b3, bc, c0, c1, c2, c3, c4, sem):
        c = lax.axis_index("c")
        s = lax.axis_index("s")
        wid = s * 2 + c
        zb = pl.multiple_of(s * 8, 8)
        for acc in (c0, c1, c2, c3, c4):
            pltpu.sync_copy(zero_h.at[pl.ds(zb, 8)], acc.at[pl.ds(zb, 8)])
        pltpu.sync_copy(ones_h, bc)
        plsc.subcore_barrier()

        def body(j, carry):
            base = pl.multiple_of(wid * per_w + j * EK, 8)
            pltpu.sync_copy(gid.at[pl.ds(base, EK)], gidx)
            for src_h, buf, acc in ((a0, b0, c0), (a1, b1, c1),
                                    (a2, b2, c2), (a3, b3, c3)):
                pltpu.sync_copy(src_h.at[pl.ds(base, EK)], buf)
                pltpu.sync_copy(buf, acc.at[gidx], add=True)
            pltpu.sync_copy(bc, c4.at[gidx], add=True)
            return carry

        lax.fori_loop(0, nblk, body, 0)
        plsc.subcore_barrier()
        for acc, out in ((c0, o0), (c1, o1), (c2, o2), (c3, o3), (c4, o4)):
            pltpu.sync_copy(acc.at[pl.ds(zb, 8)], out.at[c, pl.ds(zb, 8)])

    zero = jnp.zeros((128, W), jnp.float32)
    return k(parts[0], parts[1], parts[2], parts[3], gidp, onesb, zero)


RB = 400


def _gin_layer(h, src, dst, w1, b1, w2, b2):
    a0 = _seg_sum_128(h, src, dst, 0, 128)
    a1 = _seg_sum_128(h, src, dst, 128, 128)

    def body(h_ref, a0_ref, a1_ref, w1_ref, b1_ref, w2_ref, b2_ref, o_ref):
        hb = h_ref[...]
        agg = jnp.concatenate([a0_ref[0] + a0_ref[1], a1_ref[0] + a1_ref[1]],
                              axis=1)
        z = jnp.maximum((hb + agg) @ w1_ref[...] + b1_ref[...], 0.0)
        o_ref[...] = z @ w2_ref[...] + b2_ref[...] + hb

    part_spec = pl.BlockSpec((2, RB, 128), lambda i: (0, i, 0))
    full = lambda shp: pl.BlockSpec(shp, lambda i: (0,) * len(shp))
    return pl.pallas_call(
        body,
        grid=(N // RB,),
        in_specs=[pl.BlockSpec((RB, SUB), lambda i: (i, 0)), part_spec,
                  part_spec, full((SUB, SUB)), full((1, SUB)),
                  full((SUB, SUB)), full((1, SUB))],
        out_specs=pl.BlockSpec((RB, SUB), lambda i: (i, 0)),
        out_shape=jax.ShapeDtypeStruct((N, SUB), jnp.float32),
    )(h, a0, a1, w1, b1.reshape(1, SUB), w2, b2.reshape(1, SUB))


def _gat_prep(h, W, Ml, Mr):
    din = h.shape[1]
    width = W.shape[1]

    def body(h_ref, W_ref, Ml_ref, Mr_ref, f_ref, el_ref, er_ref):
        f = h_ref[...] @ W_ref[...]
        f_ref[...] = f
        el_ref[...] = f @ Ml_ref[...]
        er_ref[...] = f @ Mr_ref[...]

    full = lambda shp: pl.BlockSpec(shp, lambda i: (0, 0))
    row = lambda w: pl.BlockSpec((RB, w), lambda i: (i, 0))
    return pl.pallas_call(
        body,
        grid=(N // RB,),
        in_specs=[row(din), full(W.shape), full(Ml.shape), full(Mr.shape)],
        out_specs=[row(width), row(128), row(128)],
        out_shape=[jax.ShapeDtypeStruct((N, width), jnp.float32),
                   jax.ShapeDtypeStruct((N, 128), jnp.float32),
                   jax.ShapeDtypeStruct((N, 128), jnp.float32)],
    )(h, W, Ml, Mr)


def _gat_finish(h, R, b, sp, nps, nhead, dout, act):
    din = h.shape[1]
    width = nhead * dout
    halves = dout // 128

    def body(*refs):
        h_ref, R_ref, b_ref, sp_ref = refs[:4]
        np_refs = refs[4:4 + len(nps)]
        o_ref = refs[-1]
        res = h_ref[...] @ R_ref[...]
        segs = []
        for hd in range(nhead):
            sc = sp_ref[0, :, hd:hd + 1] + sp_ref[1, :, hd:hd + 1]
            inv = 1.0 / (sc + 1e-9)
            for half in range(halves):
                r = np_refs[hd * halves + half]
                segs.append((r[0] + r[1]) * inv)
        o = jnp.concatenate(segs, axis=1) + b_ref[...] + res
        if act:
            o = jnp.maximum(o, 0.0)
        o_ref[...] = o

    part_spec = pl.BlockSpec((2, RB, 128), lambda i: (0, i, 0))
    return pl.pallas_call(
        body,
        grid=(N // RB,),
        in_specs=[pl.BlockSpec((RB, din), lambda i: (i, 0)),
                  pl.BlockSpec(R.shape, lambda i: (0, 0)),
                  pl.BlockSpec((1, width), lambda i: (0, 0)),
                  part_spec] + [part_spec] * len(nps),
        out_specs=pl.BlockSpec((RB, width), lambda i: (i, 0)),
        out_shape=jax.ShapeDtypeStruct((N, width), jnp.float32),
    )(h, R, b.reshape(1, width), sp, *nps)


def _gat_layer_sc(h, src, dst, W, al, ar, b, R, nhead, dout, act):
    width = nhead * dout
    Ml = jnp.zeros((width, 128), jnp.float32)
    Mr = jnp.zeros((width, 128), jnp.float32)
    for hd in range(nhead):
        Ml = Ml.at[hd * dout:(hd + 1) * dout, hd].set(al[hd])
        Mr = Mr.at[hd * dout:(hd + 1) * dout, hd].set(ar[hd])
    feat, el128, er128 = _gat_prep(h, W, Ml, Mr)
    ex16, s_part = _gat_edge_pass1(el128, er128, src, dst)
    nps = []
    for hd in range(nhead):
        for half in range(dout // 128):
            nps.append(_gat_edge_pass2(feat, ex16, src, dst,
                                       hd * dout + half * 128, hd))
    return _gat_finish(h, R, b, s_part, nps, nhead, dout, act)


def _head(pool_parts, h_MolCLR, adM_w, adM_b, fpcat, adF_w, adF_b, p1, p2, p3):
    def body(o0, o1, o2, o3, o4, hm, amw, amb, fpc, afw, afb,
             p1_ref, p2_ref, p3_ref, out_ref):
        xsp = jnp.concatenate([o0[0, 0:B, :] + o0[1, 0:B, :],
                               o1[0, 0:B, :] + o1[1, 0:B, :]], axis=1)
        cnt = (o4[0, 0:B, :] + o4[1, 0:B, :])[:, 0:1]
        xfp = jnp.concatenate([o2[0, 0:B, :] + o2[1, 0:B, :],
                               o3[0, 0:B, :] + o3[1, 0:B, :]], axis=1)
        xfp = xfp / jnp.maximum(cnt, 1.0)
        y = (jnp.concatenate([xsp, xfp], axis=1)
             + hm[...] @ amw[...] + amb[...]
             + fpc[...] @ afw[...] + afb[...])
        z = jnp.maximum(y @ p1_ref[...], 0.0)
        z = jnp.maximum(z @ p2_ref[...], 0.0)
        out_ref[...] = z @ p3_ref[...]

    return pl.pallas_call(
        body,
        out_shape=jax.ShapeDtypeStruct((B, 11), jnp.float32),
    )(*pool_parts, h_MolCLR, adM_w, adM_b.reshape(1, MOL), fpcat,
      adF_w, adF_b.reshape(1, MOL), p1, p2, p3)


def kernel(node_feature, h_MolCLR, maccs, morgan, params, edge_index, node_subgraph, graph_ids):
    src, dst = edge_index[0], edge_index[1]
    p = params
    nsub_pad = jnp.pad(node_subgraph, (0, NP - N))
    x_sub = _embed_gather(p['embed'], nsub_pad)[:N]
    x_sub = _gin_layer(x_sub, src, dst, p['gin_w1_0'], p['gin_b1_0'], p['gin_w2_0'], p['gin_b2_0'])
    x_sub = _gin_layer(x_sub, src, dst, p['gin_w1_1'], p['gin_b1_1'], p['gin_w2_1'], p['gin_b2_1'])
    x_feat = _gat_layer_sc(node_feature, src, dst, p['gat_W_0'], p['gat_al_0'], p['gat_ar_0'], p['gat_b_0'], p['gat_R_0'], NH1, H1, True)
    x_feat = _gat_layer_sc(x_feat, src, dst, p['gat_W_1'], p['gat_al_1'], p['gat_ar_1'], p['gat_b_1'], p['gat_R_1'], NH2, O2, False)

    gidp = jnp.pad(graph_ids.astype(jnp.int32), (0, NP - N), constant_values=64)
    x2p = jnp.pad(x_sub, ((0, NP - N), (0, 0)))
    xf2p = jnp.pad(x_feat, ((0, NP - N), (0, 0)))
    onesb = jnp.ones((EK, 128), jnp.float32)
    parts = (x2p[:, :128], x2p[:, 128:], xf2p[:, :128], xf2p[:, 128:])
    pool_parts = _pool(parts, gidp, onesb)

    fpcat = jnp.concatenate([maccs, morgan], axis=1).astype(jnp.float32)
    return _head(pool_parts, h_MolCLR, p['adM_w'], p['adM_b'], fpcat,
                 p['adF_w'], p['adF_b'], p['p1'], p['p2'], p['p3'])


# speedup vs baseline: 10.9595x; 1.1859x over previous
import functools

import jax
import jax.numpy as jnp
from jax import lax
from jax.experimental import pallas as pl
from jax.experimental.pallas import tpu as pltpu
from jax.experimental.pallas import tpu_sc as plsc

N = 10000; E = 320000; B = 64; FEAT = 128; SUB = 256; MOL = 512; NF = 100000
NH1 = 5; H1 = 128; NH2 = 1; O2 = 256; SLOPE = 0.1

NP = 10240
EK = 80
_MESH = plsc.VectorSubcoreMesh(core_axis_name="c", subcore_axis_name="s")


def _tok_view(tok):
    flat = tok.reshape(-1)
    return lax.slice(flat, (0,), (128,)).reshape(1, 128)


def _seg_sum_128(table, src, dst, col0, width, tok):
    per_w = E // 32
    nblk = per_w // EK
    rows_per_sub = NP // 16

    @functools.partial(
        pl.kernel,
        mesh=_MESH,
        out_type=jax.ShapeDtypeStruct((2, NP, width), jnp.float32),
        scratch_types=[
            pltpu.VMEM((EK,), jnp.int32),
            pltpu.VMEM((EK,), jnp.int32),
            pltpu.VMEM((EK,), jnp.int32),
            pltpu.VMEM((EK,), jnp.int32),
            pltpu.VMEM((EK, width), jnp.float32),
            pltpu.VMEM((EK, width), jnp.float32),
            pltpu.VMEM_SHARED((NP, width), jnp.float32),
            pltpu.SemaphoreType.DMA,
            pltpu.SemaphoreType.DMA,
        ],
    )
    def k(tab, src_h, dst_h, zeros_h, tok_h, out,
          si0, di0, si1, di1, r0, r1, acc, sem0, sem1):
        c = lax.axis_index("c")
        s = lax.axis_index("s")
        wid = s * 2 + c
        zbase = pl.multiple_of(s * rows_per_sub, 8)
        pltpu.sync_copy(zeros_h.at[pl.ds(zbase, rows_per_sub)],
                        acc.at[pl.ds(zbase, rows_per_sub)])
        plsc.subcore_barrier()

        def load_idx(b, si, di):
            base = pl.multiple_of(wid * per_w + b * EK, 8)
            pltpu.sync_copy(src_h.at[pl.ds(base, EK)], si)
            pltpu.sync_copy(dst_h.at[pl.ds(base, EK)], di)

        def gather(si, buf, sem):
            pltpu.async_copy(tab.at[si, pl.ds(col0, width)], buf, sem)

        def consume(di, buf, sem):
            pltpu.make_async_copy(tab.at[di, pl.ds(col0, width)], buf,
                                  sem).wait()
            pltpu.sync_copy(buf, acc.at[di], add=True)

        load_idx(0, si0, di0)
        gather(si0, r0, sem0)

        def pair(jj, carry):
            b = 2 * jj
            load_idx(b + 1, si1, di1)
            gather(si1, r1, sem1)
            consume(di0, r0, sem0)
            load_idx(b + 2, si0, di0)
            gather(si0, r0, sem0)
            consume(di1, r1, sem1)
            return carry

        lax.fori_loop(0, (nblk - 1) // 2, pair, 0)
        consume(di0, r0, sem0)
        plsc.subcore_barrier()
        pltpu.sync_copy(acc.at[pl.ds(zbase, rows_per_sub)],
                        out.at[c, pl.ds(zbase, rows_per_sub)])

    zeros = jnp.zeros((NP, width), jnp.float32)
    return k(table, src, dst, zeros, _tok_view(tok))


def _embed_gather(embed, nsub_pad):
    per_w = NP // 32
    nblk = per_w // EK

    @functools.partial(
        pl.kernel,
        mesh=_MESH,
        out_type=jax.ShapeDtypeStruct((NP, SUB), jnp.float32),
        scratch_types=[
            pltpu.VMEM((EK,), jnp.int32),
            pltpu.VMEM((EK, SUB), jnp.float32),
            pltpu.SemaphoreType.DMA,
        ],
    )
    def k(emb, nsub, out, idx, buf, sem):
        c = lax.axis_index("c")
        s = lax.axis_index("s")
        wid = s * 2 + c

        def body(j, carry):
            base = pl.multiple_of(wid * per_w + j * EK, 8)
            pltpu.sync_copy(nsub.at[pl.ds(base, EK)], idx)
            pltpu.async_copy(emb.at[idx], buf, sem).wait()
            pltpu.sync_copy(buf, out.at[pl.ds(base, EK)])
            return carry

        lax.fori_loop(0, nblk, body, 0)

    return k(embed, nsub_pad)


def _gat_edge_pass1(el128, er128, src, dst, tok):
    per_w = E // 32
    nblk = per_w // EK
    rows_per_sub = NP // 16

    @functools.partial(
        pl.kernel,
        mesh=_MESH,
        out_type=[jax.ShapeDtypeStruct((E, 16), jnp.float32),
                  jax.ShapeDtypeStruct((2, NP, 128), jnp.float32)],
        scratch_types=[
            pltpu.VMEM((EK,), jnp.int32),
            pltpu.VMEM((EK,), jnp.int32),
            pltpu.VMEM((EK, 128), jnp.float32),
            pltpu.VMEM((EK, 128), jnp.float32),
            pltpu.VMEM((EK, 16), jnp.float32),
            pltpu.VMEM((EK, 128), jnp.float32),
            pltpu.VMEM_SHARED((NP, 128), jnp.float32),
            pltpu.SemaphoreType.DMA,
            pltpu.SemaphoreType.DMA,
        ],
    )
    def k(el, er, src_h, dst_h, zeros_h, tok_h, ex_out, s_out,
          sidx, didx, abuf, bbuf, exb16, exb128, sacc, sem, sem2):
        c = lax.axis_index("c")
        s = lax.axis_index("s")
        wid = s * 2 + c
        zbase = pl.multiple_of(s * rows_per_sub, 8)
        pltpu.sync_copy(zeros_h.at[pl.ds(zbase, rows_per_sub)],
                        sacc.at[pl.ds(zbase, rows_per_sub)])
        pltpu.sync_copy(zeros_h.at[pl.ds(0, EK)], exb128)
        plsc.subcore_barrier()

        def body(j, carry):
            base = pl.multiple_of(wid * per_w + j * EK, 8)
            pltpu.sync_copy(src_h.at[pl.ds(base, EK)], sidx)
            pltpu.sync_copy(dst_h.at[pl.ds(base, EK)], didx)
            cp_a = pltpu.async_copy(el.at[sidx], abuf, sem)
            cp_b = pltpu.async_copy(er.at[didx], bbuf, sem2)
            cp_a.wait()
            cp_b.wait()

            def erow(i, carry2):
                v = abuf[i, pl.ds(0, 16)] + bbuf[i, pl.ds(0, 16)]
                v = jnp.maximum(v, SLOPE * v)
                exv = jnp.exp(v)
                exb16[i, :] = exv
                exb128[i, pl.ds(0, 16)] = exv
                return carry2

            lax.fori_loop(0, EK, erow, 0)
            pltpu.sync_copy(exb16, ex_out.at[pl.ds(base, EK)])
            pltpu.sync_copy(exb128, sacc.at[didx], add=True)
            return carry

        lax.fori_loop(0, nblk, body, 0)
        plsc.subcore_barrier()
        pltpu.sync_copy(sacc.at[pl.ds(zbase, rows_per_sub)],
                        s_out.at[c, pl.ds(zbase, rows_per_sub)])

    zeros = jnp.zeros((NP, 128), jnp.float32)
    return k(el128, er128, src, dst, zeros, _tok_view(tok))


def _gat_edge_pass2(feat, ex16, src, dst, col0, hcol, tok):
    per_w = E // 32
    nblk = per_w // EK
    rows_per_sub = NP // 16
    width = 128

    @functools.partial(
        pl.kernel,
        mesh=_MESH,
        out_type=jax.ShapeDtypeStruct((2, NP, width), jnp.float32),
        scratch_types=[
            pltpu.VMEM((EK,), jnp.int32),
            pltpu.VMEM((EK,), jnp.int32),
            pltpu.VMEM((EK,), jnp.int32),
            pltpu.VMEM((EK,), jnp.int32),
            pltpu.VMEM((EK, width), jnp.float32),
            pltpu.VMEM((EK, width), jnp.float32),
            pltpu.VMEM((EK, 16), jnp.float32),
            pltpu.VMEM_SHARED((NP, width), jnp.float32),
            pltpu.SemaphoreType.DMA,
            pltpu.SemaphoreType.DMA,
        ],
    )
    def k(feat_h, ex_h, src_h, dst_h, zeros_h, tok_h, out,
          si0, di0, si1, di1, r0, r1, exb, acc, sem0, sem1):
        c = lax.axis_index("c")
        s = lax.axis_index("s")
        wid = s * 2 + c
        zbase = pl.multiple_of(s * rows_per_sub, 8)
        pltpu.sync_copy(zeros_h.at[pl.ds(zbase, rows_per_sub)],
                        acc.at[pl.ds(zbase, rows_per_sub)])
        plsc.subcore_barrier()

        def load_idx(b, si, di):
            base = pl.multiple_of(wid * per_w + b * EK, 8)
            pltpu.sync_copy(src_h.at[pl.ds(base, EK)], si)
            pltpu.sync_copy(dst_h.at[pl.ds(base, EK)], di)

        def gather(si, buf, sem):
            pltpu.async_copy(feat_h.at[si, pl.ds(col0, width)], buf, sem)

        def consume(b, di, buf, sem):
            pltpu.make_async_copy(feat_h.at[di, pl.ds(col0, width)], buf,
                                  sem).wait()
            base = pl.multiple_of(wid * per_w + b * EK, 8)
            pltpu.sync_copy(ex_h.at[pl.ds(base, EK)], exb)

            def escale(i, carry2):
                exv = exb[i, :]
                sc = jnp.full((16,), exv[hcol], jnp.float32)
                for jj in range(width // 16):
                    buf[i, pl.ds(jj * 16, 16)] = buf[i, pl.ds(jj * 16, 16)] * sc
                return carry2

            lax.fori_loop(0, EK, escale, 0)
            pltpu.sync_copy(buf, acc.at[di], add=True)

        load_idx(0, si0, di0)
        gather(si0, r0, sem0)

        def pair(jj, carry):
            b = 2 * jj
            load_idx(b + 1, si1, di1)
            gather(si1, r1, sem1)
            consume(b, di0, r0, sem0)
            load_idx(b + 2, si0, di0)
            gather(si0, r0, sem0)
            consume(b + 1, di1, r1, sem1)
            return carry

        lax.fori_loop(0, (nblk - 1) // 2, pair, 0)
        consume(nblk - 1, di0, r0, sem0)
        plsc.subcore_barrier()
        pltpu.sync_copy(acc.at[pl.ds(zbase, rows_per_sub)],
                        out.at[c, pl.ds(zbase, rows_per_sub)])

    zeros = jnp.zeros((NP, width), jnp.float32)
    return k(feat, ex16, src, dst, zeros, _tok_view(tok))


def _pool(parts, gidp, onesb):
    per_w = NP // 32
    nblk = per_w // EK
    W = 128

    @functools.partial(
        pl.kernel,
        mesh=_MESH,
        out_type=[jax.ShapeDtypeStruct((2, 128, W), jnp.float32)] * 5,
        scratch_types=(
            [pltpu.VMEM((EK,), jnp.int32)]
            + [pltpu.VMEM((EK, W), jnp.float32)] * 5
            + [pltpu.VMEM_SHARED((128, W), jnp.float32)] * 5
            + [pltpu.SemaphoreType.DMA]
        ),
    )
    def k(a0, a1, a2, a3, gid, ones_h, zero_h,
          o0, o1, o2, o3, o4,
          gidx, b0, b1, b2, b3, bc, c0, c1, c2, c3, c4, sem):
        c = lax.axis_index("c")
        s = lax.axis_index("s")
        wid = s * 2 + c
        zb = pl.multiple_of(s * 8, 8)
        for acc in (c0, c1, c2, c3, c4):
            pltpu.sync_copy(zero_h.at[pl.ds(zb, 8)], acc.at[pl.ds(zb, 8)])
        pltpu.sync_copy(ones_h, bc)
        plsc.subcore_barrier()

        def body(j, carry):
            base = pl.multiple_of(wid * per_w + j * EK, 8)
            pltpu.sync_copy(gid.at[pl.ds(base, EK)], gidx)
            for src_h, buf, acc in ((a0, b0, c0), (a1, b1, c1),
                                    (a2, b2, c2), (a3, b3, c3)):
                pltpu.sync_copy(src_h.at[pl.ds(base, EK)], buf)
                pltpu.sync_copy(buf, acc.at[gidx], add=True)
            pltpu.sync_copy(bc, c4.at[gidx], add=True)
            return carry

        lax.fori_loop(0, nblk, body, 0)
        plsc.subcore_barrier()
        for acc, out in ((c0, o0), (c1, o1), (c2, o2), (c3, o3), (c4, o4)):
            pltpu.sync_copy(acc.at[pl.ds(zb, 8)], out.at[c, pl.ds(zb, 8)])

    zero = jnp.zeros((128, W), jnp.float32)
    return k(parts[0], parts[1], parts[2], parts[3], gidp, onesb, zero)


RB = 400


def _gin_layer(h, src, dst, w1, b1, w2, b2, tok):
    a0 = _seg_sum_128(h, src, dst, 0, 128, tok)
    a1 = _seg_sum_128(h, src, dst, 128, 128, a0)

    def body(h_ref, a0_ref, a1_ref, w1_ref, b1_ref, w2_ref, b2_ref, o_ref):
        hb = h_ref[...]
        agg = jnp.concatenate([a0_ref[0] + a0_ref[1], a1_ref[0] + a1_ref[1]],
                              axis=1)
        z = jnp.maximum((hb + agg) @ w1_ref[...] + b1_ref[...], 0.0)
        o_ref[...] = z @ w2_ref[...] + b2_ref[...] + hb

    part_spec = pl.BlockSpec((2, RB, 128), lambda i: (0, i, 0))
    full = lambda shp: pl.BlockSpec(shp, lambda i: (0,) * len(shp))
    return pl.pallas_call(
        body,
        grid=(N // RB,),
        in_specs=[pl.BlockSpec((RB, SUB), lambda i: (i, 0)), part_spec,
                  part_spec, full((SUB, SUB)), full((1, SUB)),
                  full((SUB, SUB)), full((1, SUB))],
        out_specs=pl.BlockSpec((RB, SUB), lambda i: (i, 0)),
        out_shape=jax.ShapeDtypeStruct((N, SUB), jnp.float32),
    )(h, a0, a1, w1, b1.reshape(1, SUB), w2, b2.reshape(1, SUB)), a1


def _gat_prep(h, W, Ml, Mr):
    din = h.shape[1]
    width = W.shape[1]

    def body(h_ref, W_ref, Ml_ref, Mr_ref, f_ref, el_ref, er_ref):
        f = h_ref[...] @ W_ref[...]
        f_ref[...] = f
        el_ref[...] = f @ Ml_ref[...]
        er_ref[...] = f @ Mr_ref[...]

    full = lambda shp: pl.BlockSpec(shp, lambda i: (0, 0))
    row = lambda w: pl.BlockSpec((RB, w), lambda i: (i, 0))
    return pl.pallas_call(
        body,
        grid=(N // RB,),
        in_specs=[row(din), full(W.shape), full(Ml.shape), full(Mr.shape)],
        out_specs=[row(width), row(128), row(128)],
        out_shape=[jax.ShapeDtypeStruct((N, width), jnp.float32),
                   jax.ShapeDtypeStruct((N, 128), jnp.float32),
                   jax.ShapeDtypeStruct((N, 128), jnp.float32)],
    )(h, W, Ml, Mr)


def _gat_finish(h, R, b, sp, nps, nhead, dout, act):
    din = h.shape[1]
    width = nhead * dout
    halves = dout // 128

    def body(*refs):
        h_ref, R_ref, b_ref, sp_ref = refs[:4]
        np_refs = refs[4:4 + len(nps)]
        o_ref = refs[-1]
        res = h_ref[...] @ R_ref[...]
        segs = []
        for hd in range(nhead):
            sc = sp_ref[0, :, hd:hd + 1] + sp_ref[1, :, hd:hd + 1]
            inv = 1.0 / (sc + 1e-9)
            for half in range(halves):
                r = np_refs[hd * halves + half]
                segs.append((r[0] + r[1]) * inv)
        o = jnp.concatenate(segs, axis=1) + b_ref[...] + res
        if act:
            o = jnp.maximum(o, 0.0)
        o_ref[...] = o

    part_spec = pl.BlockSpec((2, RB, 128), lambda i: (0, i, 0))
    return pl.pallas_call(
        body,
        grid=(N // RB,),
        in_specs=[pl.BlockSpec((RB, din), lambda i: (i, 0)),
                  pl.BlockSpec(R.shape, lambda i: (0, 0)),
                  pl.BlockSpec((1, width), lambda i: (0, 0)),
                  part_spec] + [part_spec] * len(nps),
        out_specs=pl.BlockSpec((RB, width), lambda i: (i, 0)),
        out_shape=jax.ShapeDtypeStruct((N, width), jnp.float32),
    )(h, R, b.reshape(1, width), sp, *nps)


def _gat_layer_sc(h, src, dst, W, al, ar, b, R, nhead, dout, act, tok):
    width = nhead * dout
    Ml = jnp.zeros((width, 128), jnp.float32)
    Mr = jnp.zeros((width, 128), jnp.float32)
    for hd in range(nhead):
        Ml = Ml.at[hd * dout:(hd + 1) * dout, hd].set(al[hd])
        Mr = Mr.at[hd * dout:(hd + 1) * dout, hd].set(ar[hd])
    feat, el128, er128 = _gat_prep(h, W, Ml, Mr)
    ex16, s_part = _gat_edge_pass1(el128, er128, src, dst, tok)
    nps = []
    t = s_part
    for hd in range(nhead):
        for half in range(dout // 128):
            t = _gat_edge_pass2(feat, ex16, src, dst,
                                hd * dout + half * 128, hd, t)
            nps.append(t)
    return _gat_finish(h, R, b, s_part, nps, nhead, dout, act), t


def _head(pool_parts, h_MolCLR, adM_w, adM_b, fpcat, adF_w, adF_b, p1, p2, p3):
    def body(o0, o1, o2, o3, o4, hm, amw, amb, fpc, afw, afb,
             p1_ref, p2_ref, p3_ref, out_ref):
        xsp = jnp.concatenate([o0[0, 0:B, :] + o0[1, 0:B, :],
                               o1[0, 0:B, :] + o1[1, 0:B, :]], axis=1)
        cnt = (o4[0, 0:B, :] + o4[1, 0:B, :])[:, 0:1]
        xfp = jnp.concatenate([o2[0, 0:B, :] + o2[1, 0:B, :],
                               o3[0, 0:B, :] + o3[1, 0:B, :]], axis=1)
        xfp = xfp / jnp.maximum(cnt, 1.0)
        y = (jnp.concatenate([xsp, xfp], axis=1)
             + hm[...] @ amw[...] + amb[...]
             + fpc[...] @ afw[...] + afb[...])
        z = jnp.maximum(y @ p1_ref[...], 0.0)
        z = jnp.maximum(z @ p2_ref[...], 0.0)
        out_ref[...] = z @ p3_ref[...]

    return pl.pallas_call(
        body,
        out_shape=jax.ShapeDtypeStruct((B, 11), jnp.float32),
    )(*pool_parts, h_MolCLR, adM_w, adM_b.reshape(1, MOL), fpcat,
      adF_w, adF_b.reshape(1, MOL), p1, p2, p3)


def kernel(node_feature, h_MolCLR, maccs, morgan, params, edge_index, node_subgraph, graph_ids):
    src = edge_index[0]
    dst = edge_index[1]
    p = params
    nsub_pad = jnp.pad(node_subgraph, (0, NP - N))
    x0 = _embed_gather(p['embed'], nsub_pad)
    x_sub = x0[:N]
    x_sub, t = _gin_layer(x_sub, src, dst, p['gin_w1_0'], p['gin_b1_0'], p['gin_w2_0'], p['gin_b2_0'], x0)
    x_sub, t = _gin_layer(x_sub, src, dst, p['gin_w1_1'], p['gin_b1_1'], p['gin_w2_1'], p['gin_b2_1'], t)
    x_feat, t = _gat_layer_sc(node_feature, src, dst, p['gat_W_0'], p['gat_al_0'], p['gat_ar_0'], p['gat_b_0'], p['gat_R_0'], NH1, H1, True, t)
    x_feat, t = _gat_layer_sc(x_feat, src, dst, p['gat_W_1'], p['gat_al_1'], p['gat_ar_1'], p['gat_b_1'], p['gat_R_1'], NH2, O2, False, t)

    gidp = jnp.pad(graph_ids.astype(jnp.int32), (0, NP - N), constant_values=64)
    x2p = jnp.pad(x_sub, ((0, NP - N), (0, 0)))
    xf2p = jnp.pad(x_feat, ((0, NP - N), (0, 0)))
    onesb = jnp.ones((EK, 128), jnp.float32)
    parts = (x2p[:, :128], x2p[:, 128:], xf2p[:, :128], xf2p[:, 128:])
    pool_parts = _pool(parts, gidp, onesb)

    fpcat = jnp.concatenate([maccs, morgan], axis=1).astype(jnp.float32)
    return _head(pool_parts, h_MolCLR, p['adM_w'], p['adM_b'], fpcat,
                 p['adF_w'], p['adF_b'], p['p1'], p['p2'], p['p3'])
